# baseline (speedup 1.0000x reference)
import functools

import jax
import jax.numpy as jnp
from jax import lax
from jax.experimental import pallas as pl
from jax.experimental.pallas import tpu as pltpu
from jax.experimental.pallas import tpu_sc as plsc

NC = 2
NS = 16
NW = NC * NS

NP = 10240
ZROWS = NP // NS


def _make_deg_kernel(E, K):
    nchunk = E // (NW * K)
    mesh = plsc.VectorSubcoreMesh(
        core_axis_name="c", subcore_axis_name="s", num_cores=NC)

    @functools.partial(
        pl.kernel, mesh=mesh,
        out_type=jax.ShapeDtypeStruct((NC, NP), jnp.float32),
        scratch_types=[
            pltpu.VMEM((nchunk, K), jnp.int32),
            pltpu.VMEM((K,), jnp.float32),
            pltpu.VMEM_SHARED((NP,), jnp.float32),
            pltpu.SemaphoreType.DMA,
        ],
    )
    def deg_kernel(dst_hbm, ones_hbm, zeros_hbm, out_hbm,
                   dst_v, ones_v, acc_sh, ssem):
        cid = lax.axis_index("c")
        sid = lax.axis_index("s")
        wid = sid * NC + cid
        pltpu.sync_copy(zeros_hbm, acc_sh.at[pl.ds(sid * ZROWS, ZROWS)])
        pltpu.sync_copy(ones_hbm, ones_v)
        pltpu.sync_copy(dst_hbm.at[wid], dst_v)
        plsc.subcore_barrier()

        def body(j, carry):
            pltpu.async_copy(ones_v, acc_sh.at[dst_v.at[j]], ssem, add=True)
            return carry

        lax.fori_loop(0, nchunk, body, 0)

        def drain(j, carry):
            pltpu.make_async_copy(ones_v, acc_sh.at[dst_v.at[j]], ssem).wait()
            return carry

        lax.fori_loop(0, nchunk, drain, 0)
        plsc.subcore_barrier()
        pltpu.sync_copy(acc_sh.at[pl.ds(sid * ZROWS, ZROWS)],
                        out_hbm.at[cid, pl.ds(sid * ZROWS, ZROWS)])

    return deg_kernel


def _make_scatter_kernel(E, C, K, NBUF, NG, NR):
    nchunk = E // (NW * K)
    G = nchunk // NG
    assert nchunk == NG * G
    RW = NR // NS
    mesh = plsc.VectorSubcoreMesh(
        core_axis_name="c", subcore_axis_name="s", num_cores=NC)

    @functools.partial(
        pl.kernel, mesh=mesh,
        out_type=jax.ShapeDtypeStruct((NC, NR, C), jnp.float32),
        scratch_types=(
            [pltpu.VMEM((G * K,), jnp.int32)] * 4
            + [pltpu.VMEM((K, C), jnp.float32)] * NBUF
            + [pltpu.VMEM_SHARED((NR, C), jnp.float32)]
            + [pltpu.SemaphoreType.DMA] * (NBUF + 2)
        ),
    )
    def scatter_kernel(src_hbm, dst_hbm, y_hbm, zeros_hbm, out_hbm, *rest):
        idx = rest[:4]
        rows = rest[4:4 + NBUF]
        acc_sh = rest[4 + NBUF]
        sems = rest[4 + NBUF + 1:4 + 2 * NBUF + 1]
        isems = rest[4 + 2 * NBUF + 1:]
        cid = lax.axis_index("c")
        sid = lax.axis_index("s")
        wid = sid * NC + cid
        pltpu.sync_copy(zeros_hbm, acc_sh.at[pl.ds(sid * RW, RW)])
        pltpu.sync_copy(src_hbm.at[wid * NG], idx[0])
        pltpu.sync_copy(dst_hbm.at[wid * NG], idx[1])
        plsc.subcore_barrier()

        for g in range(NG):
            src_v = idx[(g % 2) * 2]
            dst_v = idx[(g % 2) * 2 + 1]
            if g + 1 < NG:
                nsrc = idx[((g + 1) % 2) * 2]
                ndst = idx[((g + 1) % 2) * 2 + 1]
                pltpu.async_copy(src_hbm.at[wid * NG + g + 1], nsrc, isems[0])
                pltpu.async_copy(dst_hbm.at[wid * NG + g + 1], ndst, isems[1])

            for b in range(NBUF):
                pltpu.async_copy(
                    y_hbm.at[src_v.at[pl.ds(b * K, K)]], rows[b], sems[b])

            def body(i, carry, src_v=src_v, dst_v=dst_v):
                for b in range(NBUF):
                    j = i * NBUF + b
                    pltpu.make_async_copy(
                        y_hbm.at[src_v.at[pl.ds(j * K, K)]],
                        rows[b], sems[b]).wait()
                    pltpu.sync_copy(
                        rows[b], acc_sh.at[dst_v.at[pl.ds(j * K, K)]],
                        add=True)

                    @pl.when(j + NBUF < G)
                    def _():
                        pltpu.async_copy(
                            y_hbm.at[src_v.at[pl.ds((j + NBUF) * K, K)]],
                            rows[b], sems[b])
                return carry

            lax.fori_loop(0, G // NBUF, body, 0)

            for j in range((G // NBUF) * NBUF, G):
                b = j % NBUF
                pltpu.make_async_copy(
                    y_hbm.at[src_v.at[pl.ds(j * K, K)]],
                    rows[b], sems[b]).wait()
                pltpu.sync_copy(
                    rows[b], acc_sh.at[dst_v.at[pl.ds(j * K, K)]], add=True)

            if g + 1 < NG:
                pltpu.make_async_copy(
                    src_hbm.at[wid * NG + g + 1], nsrc, isems[0]).wait()
                pltpu.make_async_copy(
                    dst_hbm.at[wid * NG + g + 1], ndst, isems[1]).wait()

        plsc.subcore_barrier()
        pltpu.sync_copy(acc_sh.at[pl.ds(sid * RW, RW)],
                        out_hbm.at[cid, pl.ds(sid * RW, RW)])

    return scatter_kernel


def _mm_scale_body(h_ref, w_ref, c0_ref, c1_ref, y_ref):
    deg = 1.0 + c0_ref[...] + c1_ref[...]
    dinv = lax.rsqrt(deg)
    xw = jnp.dot(h_ref[...], w_ref[...], preferred_element_type=jnp.float32)
    y_ref[...] = xw * dinv


def _finish_body(a0_ref, a1_ref, y_ref, c0_ref, c1_ref, b_ref, o_ref):
    deg = 1.0 + c0_ref[...] + c1_ref[...]
    dinv = lax.rsqrt(deg)
    acc = a0_ref[...] + a1_ref[...] + y_ref[...]
    o_ref[...] = jnp.maximum(acc * dinv + b_ref[...], 0.0)


def kernel(h, edges, W, b):
    N, C_IN = h.shape
    C = W.shape[1]
    E = edges.shape[1]
    K = 80
    NBUF = 4
    NG = 5
    NR = 10112
    KD = 80
    assert E % (NW * K) == 0 and E % (NW * KD) == 0

    nchunk = E // (NW * K)
    src = edges[0].reshape(NW * NG, (nchunk // NG) * K)
    dst = edges[1].reshape(NW * NG, (nchunk // NG) * K)
    dstd = edges[1].reshape(NW, E // (NW * KD), KD)
    ones_k = jnp.ones((KD,), jnp.float32)
    zeros_1d = jnp.zeros((ZROWS,), jnp.float32)
    zeros_2d = jnp.zeros((NR // NS, C), jnp.float32)

    cnt = _make_deg_kernel(E, KD)(dstd, ones_k, zeros_1d)

    BR = 2000
    grid = (N // BR,)
    c0 = cnt[0].reshape(NP, 1)
    c1 = cnt[1].reshape(NP, 1)
    y = pl.pallas_call(
        _mm_scale_body,
        grid=grid,
        in_specs=[
            pl.BlockSpec((BR, C_IN), lambda i: (i, 0)),
            pl.BlockSpec((C_IN, C), lambda i: (0, 0)),
            pl.BlockSpec((BR, 1), lambda i: (i, 0)),
            pl.BlockSpec((BR, 1), lambda i: (i, 0)),
        ],
        out_specs=pl.BlockSpec((BR, C), lambda i: (i, 0)),
        out_shape=jax.ShapeDtypeStruct((N, C), jnp.float32),
    )(h, W, c0, c1)

    acc = _make_scatter_kernel(E, C, K, NBUF, NG, NR)(src, dst, y, zeros_2d)

    out = pl.pallas_call(
        _finish_body,
        grid=grid,
        in_specs=[
            pl.BlockSpec((BR, C), lambda i: (i, 0)),
            pl.BlockSpec((BR, C), lambda i: (i, 0)),
            pl.BlockSpec((BR, C), lambda i: (i, 0)),
            pl.BlockSpec((BR, 1), lambda i: (i, 0)),
            pl.BlockSpec((BR, 1), lambda i: (i, 0)),
            pl.BlockSpec((1, C), lambda i: (0, 0)),
        ],
        out_specs=pl.BlockSpec((BR, C), lambda i: (i, 0)),
        out_shape=jax.ShapeDtypeStruct((N, C), jnp.float32),
    )(acc[0], acc[1], y, c0, c1, b.reshape(1, C))
    return out


# speedup vs baseline: 1.0070x; 1.0070x over previous
import functools

import jax
import jax.numpy as jnp
from jax import lax
from jax.experimental import pallas as pl
from jax.experimental.pallas import tpu as pltpu
from jax.experimental.pallas import tpu_sc as plsc

NC = 2
NS = 16
NW = NC * NS

NP = 10240
ZROWS = NP // NS


def _make_deg_kernel(E, K):
    nchunk = E // (NW * K)
    mesh = plsc.VectorSubcoreMesh(
        core_axis_name="c", subcore_axis_name="s", num_cores=NC)

    @functools.partial(
        pl.kernel, mesh=mesh,
        out_type=jax.ShapeDtypeStruct((NC, NP), jnp.float32),
        scratch_types=[
            pltpu.VMEM((nchunk, K), jnp.int32),
            pltpu.VMEM((K,), jnp.float32),
            pltpu.VMEM_SHARED((NP,), jnp.float32),
            pltpu.SemaphoreType.DMA,
        ],
    )
    def deg_kernel(dst_hbm, ones_hbm, zeros_hbm, out_hbm,
                   dst_v, ones_v, acc_sh, ssem):
        cid = lax.axis_index("c")
        sid = lax.axis_index("s")
        wid = sid * NC + cid
        pltpu.sync_copy(zeros_hbm, acc_sh.at[pl.ds(sid * ZROWS, ZROWS)])
        pltpu.sync_copy(ones_hbm, ones_v)
        pltpu.sync_copy(dst_hbm.at[wid], dst_v)
        plsc.subcore_barrier()

        def body(j, carry):
            pltpu.async_copy(ones_v, acc_sh.at[dst_v.at[j]], ssem, add=True)
            return carry

        lax.fori_loop(0, nchunk, body, 0)

        def drain(j, carry):
            pltpu.make_async_copy(ones_v, acc_sh.at[dst_v.at[j]], ssem).wait()
            return carry

        lax.fori_loop(0, nchunk, drain, 0)
        plsc.subcore_barrier()
        pltpu.sync_copy(acc_sh.at[pl.ds(sid * ZROWS, ZROWS)],
                        out_hbm.at[cid, pl.ds(sid * ZROWS, ZROWS)])

    return deg_kernel


def _make_scatter_kernel(E, C, K, NBUF, NG, NR):
    nchunk = E // (NW * K)
    G = nchunk // NG
    assert nchunk == NG * G
    RW = NR // NS
    mesh = plsc.VectorSubcoreMesh(
        core_axis_name="c", subcore_axis_name="s", num_cores=NC)

    @functools.partial(
        pl.kernel, mesh=mesh,
        out_type=jax.ShapeDtypeStruct((NC, NR, C), jnp.float32),
        scratch_types=(
            [pltpu.VMEM((G * K,), jnp.int32)] * 4
            + [pltpu.VMEM((K, C), jnp.float32)] * NBUF
            + [pltpu.VMEM_SHARED((NR, C), jnp.float32)]
            + [pltpu.SemaphoreType.DMA] * (NBUF + 2)
        ),
    )
    def scatter_kernel(src_hbm, dst_hbm, y_hbm, zeros_hbm, out_hbm, *rest):
        idx = rest[:4]
        rows = rest[4:4 + NBUF]
        acc_sh = rest[4 + NBUF]
        sems = rest[4 + NBUF + 1:4 + 2 * NBUF + 1]
        isems = rest[4 + 2 * NBUF + 1:]
        cid = lax.axis_index("c")
        sid = lax.axis_index("s")
        wid = sid * NC + cid
        pltpu.sync_copy(zeros_hbm, acc_sh.at[pl.ds(sid * RW, RW)])
        pltpu.sync_copy(src_hbm.at[wid * NG], idx[0])
        pltpu.sync_copy(dst_hbm.at[wid * NG], idx[1])
        plsc.subcore_barrier()

        for g in range(NG):
            src_v = idx[(g % 2) * 2]
            dst_v = idx[(g % 2) * 2 + 1]
            if g + 1 < NG:
                nsrc = idx[((g + 1) % 2) * 2]
                ndst = idx[((g + 1) % 2) * 2 + 1]
                pltpu.async_copy(src_hbm.at[wid * NG + g + 1], nsrc, isems[0])
                pltpu.async_copy(dst_hbm.at[wid * NG + g + 1], ndst, isems[1])

            for b in range(NBUF):
                pltpu.async_copy(
                    y_hbm.at[src_v.at[pl.ds(b * K, K)]], rows[b], sems[b])

            def body(i, carry, src_v=src_v, dst_v=dst_v):
                for b in range(NBUF):
                    j = i * NBUF + b
                    pltpu.make_async_copy(
                        y_hbm.at[src_v.at[pl.ds(j * K, K)]],
                        rows[b], sems[b]).wait()
                    pltpu.sync_copy(
                        rows[b], acc_sh.at[dst_v.at[pl.ds(j * K, K)]],
                        add=True)

                    @pl.when(j + NBUF < G)
                    def _():
                        pltpu.async_copy(
                            y_hbm.at[src_v.at[pl.ds((j + NBUF) * K, K)]],
                            rows[b], sems[b])
                return carry

            lax.fori_loop(0, G // NBUF, body, 0)

            for j in range((G // NBUF) * NBUF, G):
                b = j % NBUF
                pltpu.make_async_copy(
                    y_hbm.at[src_v.at[pl.ds(j * K, K)]],
                    rows[b], sems[b]).wait()
                pltpu.sync_copy(
                    rows[b], acc_sh.at[dst_v.at[pl.ds(j * K, K)]], add=True)

            if g + 1 < NG:
                pltpu.make_async_copy(
                    src_hbm.at[wid * NG + g + 1], nsrc, isems[0]).wait()
                pltpu.make_async_copy(
                    dst_hbm.at[wid * NG + g + 1], ndst, isems[1]).wait()

        plsc.subcore_barrier()
        pltpu.sync_copy(acc_sh.at[pl.ds(sid * RW, RW)],
                        out_hbm.at[cid, pl.ds(sid * RW, RW)])

    return scatter_kernel


def _mm_scale_body(h_ref, w_ref, c0_ref, c1_ref, y_ref):
    deg = 1.0 + c0_ref[...] + c1_ref[...]
    dinv = lax.rsqrt(deg)
    xw = jnp.dot(h_ref[...], w_ref[...], preferred_element_type=jnp.float32)
    y_ref[...] = xw * dinv


def _finish_body(a0_ref, a1_ref, y_ref, c0_ref, c1_ref, b_ref, o_ref):
    deg = 1.0 + c0_ref[...] + c1_ref[...]
    dinv = lax.rsqrt(deg)
    acc = a0_ref[...] + a1_ref[...] + y_ref[...]
    o_ref[...] = jnp.maximum(acc * dinv + b_ref[...], 0.0)


def kernel(h, edges, W, b):
    N, C_IN = h.shape
    C = W.shape[1]
    E = edges.shape[1]
    K = 40
    NBUF = 8
    NG = 5
    NR = 10112
    KD = 80
    assert E % (NW * K) == 0 and E % (NW * KD) == 0

    nchunk = E // (NW * K)
    src = edges[0].reshape(NW * NG, (nchunk // NG) * K)
    dst = edges[1].reshape(NW * NG, (nchunk // NG) * K)
    dstd = edges[1].reshape(NW, E // (NW * KD), KD)
    ones_k = jnp.ones((KD,), jnp.float32)
    zeros_1d = jnp.zeros((ZROWS,), jnp.float32)
    zeros_2d = jnp.zeros((NR // NS, C), jnp.float32)

    cnt = _make_deg_kernel(E, KD)(dstd, ones_k, zeros_1d)

    BR = 2000
    grid = (N // BR,)
    c0 = cnt[0].reshape(NP, 1)
    c1 = cnt[1].reshape(NP, 1)
    y = pl.pallas_call(
        _mm_scale_body,
        grid=grid,
        in_specs=[
            pl.BlockSpec((BR, C_IN), lambda i: (i, 0)),
            pl.BlockSpec((C_IN, C), lambda i: (0, 0)),
            pl.BlockSpec((BR, 1), lambda i: (i, 0)),
            pl.BlockSpec((BR, 1), lambda i: (i, 0)),
        ],
        out_specs=pl.BlockSpec((BR, C), lambda i: (i, 0)),
        out_shape=jax.ShapeDtypeStruct((N, C), jnp.float32),
    )(h, W, c0, c1)

    acc = _make_scatter_kernel(E, C, K, NBUF, NG, NR)(src, dst, y, zeros_2d)

    out = pl.pallas_call(
        _finish_body,
        grid=grid,
        in_specs=[
            pl.BlockSpec((BR, C), lambda i: (i, 0)),
            pl.BlockSpec((BR, C), lambda i: (i, 0)),
            pl.BlockSpec((BR, C), lambda i: (i, 0)),
            pl.BlockSpec((BR, 1), lambda i: (i, 0)),
            pl.BlockSpec((BR, 1), lambda i: (i, 0)),
            pl.BlockSpec((1, C), lambda i: (0, 0)),
        ],
        out_specs=pl.BlockSpec((BR, C), lambda i: (i, 0)),
        out_shape=jax.ShapeDtypeStruct((N, C), jnp.float32),
    )(acc[0], acc[1], y, c0, c1, b.reshape(1, C))
    return out


# speedup vs baseline: 1.0494x; 1.0421x over previous
import functools

import jax
import jax.numpy as jnp
from jax import lax
from jax.experimental import pallas as pl
from jax.experimental.pallas import tpu as pltpu
from jax.experimental.pallas import tpu_sc as plsc

NC = 2
NS = 16
NW = NC * NS

NP = 10240
ZROWS = NP // NS


def _make_deg_kernel(E, K):
    nchunk = E // (NW * K)
    mesh = plsc.VectorSubcoreMesh(
        core_axis_name="c", subcore_axis_name="s", num_cores=NC)

    @functools.partial(
        pl.kernel, mesh=mesh,
        out_type=jax.ShapeDtypeStruct((NC, NP), jnp.float32),
        scratch_types=[
            pltpu.VMEM((nchunk, K), jnp.int32),
            pltpu.VMEM((K,), jnp.float32),
            pltpu.VMEM_SHARED((NP,), jnp.float32),
            pltpu.SemaphoreType.DMA,
        ],
    )
    def deg_kernel(dst_hbm, ones_hbm, zeros_hbm, out_hbm,
                   dst_v, ones_v, acc_sh, ssem):
        cid = lax.axis_index("c")
        sid = lax.axis_index("s")
        wid = sid * NC + cid
        pltpu.sync_copy(zeros_hbm, acc_sh.at[pl.ds(sid * ZROWS, ZROWS)])
        pltpu.sync_copy(ones_hbm, ones_v)
        pltpu.sync_copy(dst_hbm.at[wid], dst_v)
        plsc.subcore_barrier()

        def body(j, carry):
            pltpu.async_copy(ones_v, acc_sh.at[dst_v.at[j]], ssem, add=True)
            return carry

        lax.fori_loop(0, nchunk, body, 0)

        def drain(j, carry):
            pltpu.make_async_copy(ones_v, acc_sh.at[dst_v.at[j]], ssem).wait()
            return carry

        lax.fori_loop(0, nchunk, drain, 0)
        plsc.subcore_barrier()
        pltpu.sync_copy(acc_sh.at[pl.ds(sid * ZROWS, ZROWS)],
                        out_hbm.at[cid, pl.ds(sid * ZROWS, ZROWS)])

    return deg_kernel


def _make_scatter_kernel(E, C, K, NBUF):
    nchunk = E // (NW * K)
    mesh = plsc.VectorSubcoreMesh(
        core_axis_name="c", subcore_axis_name="s", num_cores=NC)

    @functools.partial(
        pl.kernel, mesh=mesh,
        out_type=jax.ShapeDtypeStruct((NC, NP, C), jnp.float32),
        scratch_types=(
            [pltpu.VMEM((nchunk * K,), jnp.int32)] * 2
            + [pltpu.VMEM((K, C), jnp.float32)] * NBUF
            + [pltpu.VMEM_SHARED((NP, C), jnp.float32)]
            + [pltpu.SemaphoreType.DMA] * NBUF
        ),
    )
    def scatter_kernel(src_hbm, dst_hbm, y_hbm, zeros_hbm, out_hbm,
                       src_v, dst_v, *rest):
        rows = rest[:NBUF]
        acc_sh = rest[NBUF]
        sems = rest[NBUF + 1:]
        cid = lax.axis_index("c")
        sid = lax.axis_index("s")
        wid = sid * NC + cid
        pltpu.sync_copy(zeros_hbm, acc_sh.at[pl.ds(sid * ZROWS, ZROWS)])
        pltpu.sync_copy(src_hbm.at[wid], src_v)
        pltpu.sync_copy(dst_hbm.at[wid], dst_v)
        plsc.subcore_barrier()

        for b in range(NBUF):
            pltpu.async_copy(
                y_hbm.at[src_v.at[pl.ds(b * K, K)]], rows[b], sems[b])

        def body(g, carry):
            for b in range(NBUF):
                j = g * NBUF + b
                pltpu.make_async_copy(
                    y_hbm.at[src_v.at[pl.ds(j * K, K)]],
                    rows[b], sems[b]).wait()
                pltpu.sync_copy(
                    rows[b], acc_sh.at[dst_v.at[pl.ds(j * K, K)]], add=True)

                @pl.when(j + NBUF < nchunk)
                def _():
                    pltpu.async_copy(
                        y_hbm.at[src_v.at[pl.ds((j + NBUF) * K, K)]],
                        rows[b], sems[b])
            return carry

        lax.fori_loop(0, nchunk // NBUF, body, 0)

        for j in range((nchunk // NBUF) * NBUF, nchunk):
            b = j % NBUF
            pltpu.make_async_copy(
                y_hbm.at[src_v.at[pl.ds(j * K, K)]], rows[b], sems[b]).wait()
            pltpu.sync_copy(
                rows[b], acc_sh.at[dst_v.at[pl.ds(j * K, K)]], add=True)

        plsc.subcore_barrier()
        pltpu.sync_copy(acc_sh.at[pl.ds(sid * ZROWS, ZROWS)],
                        out_hbm.at[cid, pl.ds(sid * ZROWS, ZROWS)])

    return scatter_kernel


def _mm_scale_body(h_ref, w_ref, c0_ref, c1_ref, y_ref):
    deg = 1.0 + c0_ref[...] + c1_ref[...]
    dinv = lax.rsqrt(deg)
    xw = jnp.dot(h_ref[...], w_ref[...], preferred_element_type=jnp.float32)
    y_ref[...] = xw * dinv


def _finish_body(a0_ref, a1_ref, y_ref, c0_ref, c1_ref, b_ref, o_ref):
    deg = 1.0 + c0_ref[...] + c1_ref[...]
    dinv = lax.rsqrt(deg)
    acc = a0_ref[...] + a1_ref[...] + y_ref[...]
    o_ref[...] = jnp.maximum(acc * dinv + b_ref[...], 0.0)


def kernel(h, edges, W, b):
    N, C_IN = h.shape
    C = W.shape[1]
    E = edges.shape[1]
    K = 40
    NBUF = 5
    KD = 80
    assert E % (NW * K) == 0 and E % (NW * KD) == 0

    nchunk = E // (NW * K)
    src = edges[0].reshape(NW, nchunk * K)
    dst = edges[1].reshape(NW, nchunk * K)
    dstd = edges[1].reshape(NW, E // (NW * KD), KD)
    ones_k = jnp.ones((KD,), jnp.float32)
    zeros_1d = jnp.zeros((ZROWS,), jnp.float32)
    zeros_2d = jnp.zeros((ZROWS, C), jnp.float32)

    cnt = _make_deg_kernel(E, KD)(dstd, ones_k, zeros_1d)

    BR = 2000
    grid = (N // BR,)
    c0 = cnt[0].reshape(NP, 1)
    c1 = cnt[1].reshape(NP, 1)
    y = pl.pallas_call(
        _mm_scale_body,
        grid=grid,
        in_specs=[
            pl.BlockSpec((BR, C_IN), lambda i: (i, 0)),
            pl.BlockSpec((C_IN, C), lambda i: (0, 0)),
            pl.BlockSpec((BR, 1), lambda i: (i, 0)),
            pl.BlockSpec((BR, 1), lambda i: (i, 0)),
        ],
        out_specs=pl.BlockSpec((BR, C), lambda i: (i, 0)),
        out_shape=jax.ShapeDtypeStruct((N, C), jnp.float32),
    )(h, W, c0, c1)

    acc = _make_scatter_kernel(E, C, K, NBUF)(src, dst, y, zeros_2d)

    out = pl.pallas_call(
        _finish_body,
        grid=grid,
        in_specs=[
            pl.BlockSpec((BR, C), lambda i: (i, 0)),
            pl.BlockSpec((BR, C), lambda i: (i, 0)),
            pl.BlockSpec((BR, C), lambda i: (i, 0)),
            pl.BlockSpec((BR, 1), lambda i: (i, 0)),
            pl.BlockSpec((BR, 1), lambda i: (i, 0)),
            pl.BlockSpec((1, C), lambda i: (0, 0)),
        ],
        out_specs=pl.BlockSpec((BR, C), lambda i: (i, 0)),
        out_shape=jax.ShapeDtypeStruct((N, C), jnp.float32),
    )(acc[0], acc[1], y, c0, c1, b.reshape(1, C))
    return out
